# baseline (speedup 1.0000x reference)
import dataclasses
import functools

import jax
import jax.numpy as jnp
from jax import lax
from jax.experimental import pallas as pl
from jax.experimental.pallas import tpu as pltpu
from jax.experimental.pallas import tpu_sc as plsc

D = 128
W = D // 2
L = 16
NC, NS = 2, 16
NW = NC * NS
C = 128
CHUNKS0 = 94
CHUNKS1 = 66


@functools.cache
def _dot_kernel(E_pad):
    assert E_pad == NS * (CHUNKS0 + CHUNKS1) * C

    mesh = plsc.VectorSubcoreMesh(core_axis_name="c", subcore_axis_name="s")

    cp = pltpu.CompilerParams(use_tc_tiling_on_sc=False)
    if "needs_layout_passes" in pltpu.CompilerParams.__dataclass_fields__:
        cp = dataclasses.replace(cp, needs_layout_passes=False)

    @functools.partial(
        pl.kernel,
        mesh=mesh,
        compiler_params=cp,
        out_type=jax.ShapeDtypeStruct((E_pad,), jnp.float32),
        scratch_types=[
            pltpu.VMEM((C,), jnp.int32),
            pltpu.VMEM((C,), jnp.int32),
            pltpu.VMEM((C, W), jnp.int32),
            pltpu.VMEM((C, W), jnp.int32),
            pltpu.VMEM((C,), jnp.float32),
            pltpu.SemaphoreType.DMA,
            pltpu.SemaphoreType.DMA,
        ],
    )
    def k(h_hbm, src_hbm, dst_hbm, out_hbm,
          sidx, didx, srows, drows, ovec, sem_s, sem_d):
        cid = lax.axis_index("c")
        sid = lax.axis_index("s")

        def run_chunk(b):
            pltpu.sync_copy(src_hbm.at[pl.ds(b, C)], sidx)
            pltpu.sync_copy(dst_hbm.at[pl.ds(b, C)], didx)
            cps = pltpu.async_copy(h_hbm.at[sidx], srows, sem_s)
            cpd = pltpu.async_copy(h_hbm.at[didx], drows, sem_d)
            cps.wait()
            cpd.wait()

            lane = lax.iota(jnp.int32, L)
            last = lane == (L - 1)

            @pl.loop(0, C // L)
            def _grp(g):
                e0 = g * L
                e0v = jnp.full((L,), 0, jnp.int32) + e0
                for j in range(L):
                    e = e0 + j
                    p = None
                    for kk in range(W // L):
                        sv = plsc.bitcast(srows[e, pl.ds(kk * L, L)],
                                          jnp.bfloat16)
                        dv = plsc.bitcast(drows[e, pl.ds(kk * L, L)],
                                          jnp.bfloat16)
                        sa, sb = plsc.unpack(
                            sv, format=plsc.PackFormat.INTERLEAVED)
                        da, db = plsc.unpack(
                            dv, format=plsc.PackFormat.INTERLEAVED)
                        q = sa * da + sb * db
                        p = q if p is None else p + q
                    ps = lax.cumsum(p, axis=0)
                    plsc.store_scatter(ovec, [e0v + j], ps, mask=last)

            pltpu.sync_copy(ovec, out_hbm.at[pl.ds(b, C)])

        @pl.when(cid == 0)
        def _():
            base = sid * (CHUNKS0 * C)

            @pl.loop(0, CHUNKS0)
            def _chunk(t):
                run_chunk(base + t * C)

        @pl.when(cid == 1)
        def _():
            base = NS * (CHUNKS0 * C) + sid * (CHUNKS1 * C)

            @pl.loop(0, CHUNKS1)
            def _chunk(t):
                run_chunk(base + t * C)

    return k


def kernel(h, edge_index):
    E = edge_index.shape[1]
    src = edge_index[0].astype(jnp.int32)
    dst = edge_index[1].astype(jnp.int32)

    E_pad = NS * (CHUNKS0 + CHUNKS1) * C
    assert E_pad >= E
    if E_pad != E:
        pad = E_pad - E
        zeros = jnp.zeros((pad,), jnp.int32)
        src = jnp.concatenate([src, zeros])
        dst = jnp.concatenate([dst, zeros])

    h32 = jax.lax.bitcast_convert_type(
        h.astype(jnp.bfloat16).reshape(h.shape[0], W, 2), jnp.int32)
    out = _dot_kernel(E_pad)(h32, src, dst)
    return out[:E].reshape(E, 1)


# speedup vs baseline: 1.8306x; 1.8306x over previous
import dataclasses
import functools

import jax
import jax.numpy as jnp
from jax import lax
from jax.experimental import pallas as pl
from jax.experimental.pallas import tpu as pltpu
from jax.experimental.pallas import tpu_sc as plsc

D = 128
W = D // 2
L = 16
NC, NS = 2, 16
NW = NC * NS
C = 128


@functools.cache
def _dot_kernel(E_pad, n_rows):
    per_w = E_pad // NW
    n_chunks = per_w // C
    assert n_chunks * C * NW == E_pad

    rows_per_tile = ((n_rows // NS) // 8) * 8
    tail_rows = n_rows - (NS - 1) * rows_per_tile

    mesh = plsc.VectorSubcoreMesh(core_axis_name="c", subcore_axis_name="s")

    cp = pltpu.CompilerParams(use_tc_tiling_on_sc=False)
    if "needs_layout_passes" in pltpu.CompilerParams.__dataclass_fields__:
        cp = dataclasses.replace(cp, needs_layout_passes=False)

    @functools.partial(
        pl.kernel,
        mesh=mesh,
        compiler_params=cp,
        out_type=jax.ShapeDtypeStruct((E_pad,), jnp.float32),
        scratch_types=[
            pltpu.VMEM_SHARED((n_rows, W), jnp.int32),
            pltpu.VMEM((C,), jnp.int32),
            pltpu.VMEM((C,), jnp.int32),
            pltpu.VMEM((C, W), jnp.int32),
            pltpu.VMEM((C, W), jnp.int32),
            pltpu.VMEM((C,), jnp.float32),
            pltpu.SemaphoreType.DMA,
            pltpu.SemaphoreType.DMA,
        ],
    )
    def k(h_hbm, src_hbm, dst_hbm, out_hbm,
          h_sh, sidx, didx, srows, drows, ovec, sem_s, sem_d):
        cid = lax.axis_index("c")
        sid = lax.axis_index("s")
        wid = cid * NS + sid
        base = wid * per_w

        @pl.when(sid < NS - 1)
        def _():
            r0 = sid * rows_per_tile
            pltpu.sync_copy(h_hbm.at[pl.ds(r0, rows_per_tile)],
                            h_sh.at[pl.ds(r0, rows_per_tile)])

        @pl.when(sid == NS - 1)
        def _():
            r0 = (NS - 1) * rows_per_tile
            pltpu.sync_copy(h_hbm.at[pl.ds(r0, tail_rows)],
                            h_sh.at[pl.ds(r0, tail_rows)])

        plsc.subcore_barrier()

        def run_chunk(b):
            pltpu.sync_copy(src_hbm.at[pl.ds(b, C)], sidx)
            pltpu.sync_copy(dst_hbm.at[pl.ds(b, C)], didx)
            cps = pltpu.async_copy(h_sh.at[sidx], srows, sem_s)
            cpd = pltpu.async_copy(h_sh.at[didx], drows, sem_d)
            cps.wait()
            cpd.wait()

            lane = lax.iota(jnp.int32, L)
            last = lane == (L - 1)

            @pl.loop(0, C // L)
            def _grp(g):
                e0 = g * L
                e0v = jnp.full((L,), 0, jnp.int32) + e0
                for j in range(L):
                    e = e0 + j
                    p = None
                    for kk in range(W // L):
                        sv = plsc.bitcast(srows[e, pl.ds(kk * L, L)],
                                          jnp.bfloat16)
                        dv = plsc.bitcast(drows[e, pl.ds(kk * L, L)],
                                          jnp.bfloat16)
                        sa, sb = plsc.unpack(
                            sv, format=plsc.PackFormat.INTERLEAVED)
                        da, db = plsc.unpack(
                            dv, format=plsc.PackFormat.INTERLEAVED)
                        q = sa * da + sb * db
                        p = q if p is None else p + q
                    ps = lax.cumsum(p, axis=0)
                    plsc.store_scatter(ovec, [e0v + j], ps, mask=last)

            pltpu.sync_copy(ovec, out_hbm.at[pl.ds(b, C)])

        @pl.loop(0, n_chunks)
        def _chunk(t):
            run_chunk(base + t * C)

    return k


def kernel(h, edge_index):
    E = edge_index.shape[1]
    src = edge_index[0].astype(jnp.int32)
    dst = edge_index[1].astype(jnp.int32)

    step = NW * C
    E_pad = ((E + step - 1) // step) * step
    if E_pad != E:
        pad = E_pad - E
        zeros = jnp.zeros((pad,), jnp.int32)
        src = jnp.concatenate([src, zeros])
        dst = jnp.concatenate([dst, zeros])

    h32 = jax.lax.bitcast_convert_type(
        h.astype(jnp.bfloat16).reshape(h.shape[0], W, 2), jnp.int32)
    out = _dot_kernel(E_pad, h.shape[0])(h32, src, dst)
    return out[:E].reshape(E, 1)


# speedup vs baseline: 2.0953x; 1.1446x over previous
import dataclasses
import functools

import jax
import jax.numpy as jnp
from jax import lax
from jax.experimental import pallas as pl
from jax.experimental.pallas import tpu as pltpu
from jax.experimental.pallas import tpu_sc as plsc

D = 128
W = D // 2
L = 16
NC, NS = 2, 16
NW = NC * NS
C = 128


@functools.cache
def _dot_kernel(E_pad, n_rows):
    per_w = E_pad // NW
    n_chunks = per_w // C
    assert n_chunks * C * NW == E_pad

    rows_per_tile = ((n_rows // NS) // 8) * 8
    tail_rows = n_rows - (NS - 1) * rows_per_tile

    mesh = plsc.VectorSubcoreMesh(core_axis_name="c", subcore_axis_name="s")

    cp = pltpu.CompilerParams(use_tc_tiling_on_sc=False)
    if "needs_layout_passes" in pltpu.CompilerParams.__dataclass_fields__:
        cp = dataclasses.replace(cp, needs_layout_passes=False)

    @functools.partial(
        pl.kernel,
        mesh=mesh,
        compiler_params=cp,
        out_type=jax.ShapeDtypeStruct((E_pad,), jnp.float32),
        scratch_types=[
            pltpu.VMEM_SHARED((n_rows, W), jnp.int32),
            pltpu.VMEM((C,), jnp.int32),
            pltpu.VMEM((C,), jnp.int32),
            pltpu.VMEM((C,), jnp.int32),
            pltpu.VMEM((C,), jnp.int32),
            pltpu.VMEM((C, W), jnp.int32),
            pltpu.VMEM((C, W), jnp.int32),
            pltpu.VMEM((C, W), jnp.int32),
            pltpu.VMEM((C, W), jnp.int32),
            pltpu.VMEM((C,), jnp.float32),
            pltpu.SemaphoreType.DMA,
            pltpu.SemaphoreType.DMA,
            pltpu.SemaphoreType.DMA,
            pltpu.SemaphoreType.DMA,
        ],
    )
    def k(h_hbm, src_hbm, dst_hbm, out_hbm,
          h_sh, sidx0, didx0, sidx1, didx1,
          srows0, drows0, srows1, drows1, ovec,
          sem_s0, sem_d0, sem_s1, sem_d1):
        cid = lax.axis_index("c")
        sid = lax.axis_index("s")
        wid = cid * NS + sid
        base = wid * per_w

        @pl.when(sid < NS - 1)
        def _():
            r0 = sid * rows_per_tile
            pltpu.sync_copy(h_hbm.at[pl.ds(r0, rows_per_tile)],
                            h_sh.at[pl.ds(r0, rows_per_tile)])

        @pl.when(sid == NS - 1)
        def _():
            r0 = (NS - 1) * rows_per_tile
            pltpu.sync_copy(h_hbm.at[pl.ds(r0, tail_rows)],
                            h_sh.at[pl.ds(r0, tail_rows)])

        plsc.subcore_barrier()

        bufs = ((sidx0, didx0, srows0, drows0, sem_s0, sem_d0),
                (sidx1, didx1, srows1, drows1, sem_s1, sem_d1))

        def issue(t, b):
            si, di, sr, dr, ss, sd = bufs[b]
            off = base + t * C
            pltpu.sync_copy(src_hbm.at[pl.ds(off, C)], si)
            pltpu.sync_copy(dst_hbm.at[pl.ds(off, C)], di)
            pltpu.async_copy(h_sh.at[si], sr, ss)
            pltpu.async_copy(h_sh.at[di], dr, sd)

        def drain(b):
            si, di, sr, dr, ss, sd = bufs[b]
            pltpu.make_async_copy(h_sh.at[si], sr, ss).wait()
            pltpu.make_async_copy(h_sh.at[di], dr, sd).wait()

        def compute(t, b):
            sr, dr = bufs[b][2], bufs[b][3]
            lane = lax.iota(jnp.int32, L)
            last = lane == (L - 1)

            @pl.loop(0, C // L)
            def _grp(g):
                e0 = g * L
                e0v = jnp.full((L,), 0, jnp.int32) + e0
                for j in range(L):
                    e = e0 + j
                    p = None
                    for kk in range(W // L):
                        sv = plsc.bitcast(sr[e, pl.ds(kk * L, L)],
                                          jnp.bfloat16)
                        dv = plsc.bitcast(dr[e, pl.ds(kk * L, L)],
                                          jnp.bfloat16)
                        sa, sb = plsc.unpack(
                            sv, format=plsc.PackFormat.INTERLEAVED)
                        da, db = plsc.unpack(
                            dv, format=plsc.PackFormat.INTERLEAVED)
                        q = sa * da + sb * db
                        p = q if p is None else p + q
                    ps = lax.cumsum(p, axis=0)
                    plsc.store_scatter(ovec, [e0v + j], ps, mask=last)

            pltpu.sync_copy(ovec, out_hbm.at[pl.ds(base + t * C, C)])

        assert n_chunks % 2 == 0
        issue(0, 0)

        @pl.loop(0, n_chunks, step=2)
        def _chunk(t):
            issue(t + 1, 1)
            drain(0)
            compute(t, 0)

            @pl.when(t + 2 < n_chunks)
            def _():
                issue(t + 2, 0)

            drain(1)
            compute(t + 1, 1)

    return k


def kernel(h, edge_index):
    E = edge_index.shape[1]
    src = edge_index[0].astype(jnp.int32)
    dst = edge_index[1].astype(jnp.int32)

    step = NW * C * 2
    E_pad = ((E + step - 1) // step) * step
    if E_pad != E:
        pad = E_pad - E
        zeros = jnp.zeros((pad,), jnp.int32)
        src = jnp.concatenate([src, zeros])
        dst = jnp.concatenate([dst, zeros])

    h32 = jax.lax.bitcast_convert_type(
        h.astype(jnp.bfloat16).reshape(h.shape[0], W, 2), jnp.int32)
    out = _dot_kernel(E_pad, h.shape[0])(h32, src, dst)
    return out[:E].reshape(E, 1)


# speedup vs baseline: 2.6931x; 1.2853x over previous
import dataclasses
import functools

import jax
import jax.numpy as jnp
from jax import lax
from jax.experimental import pallas as pl
from jax.experimental.pallas import tpu as pltpu
from jax.experimental.pallas import tpu_sc as plsc

D = 128
W = D // 2
L = 16
NC, NS = 2, 16
NW = NC * NS
C = 128


@functools.cache
def _dot_kernel(E_pad, n_rows):
    per_w = E_pad // NW
    n_chunks = per_w // C
    assert n_chunks % 2 == 0 and n_chunks * C * NW == E_pad

    rows_per_tile = ((n_rows // NS) // 8) * 8
    tail_rows = n_rows - (NS - 1) * rows_per_tile

    mesh = plsc.VectorSubcoreMesh(core_axis_name="c", subcore_axis_name="s")

    cp = pltpu.CompilerParams(use_tc_tiling_on_sc=False)
    if "needs_layout_passes" in pltpu.CompilerParams.__dataclass_fields__:
        cp = dataclasses.replace(cp, needs_layout_passes=False)

    @functools.partial(
        pl.kernel,
        mesh=mesh,
        compiler_params=cp,
        out_type=jax.ShapeDtypeStruct((E_pad,), jnp.float32),
        scratch_types=[
            pltpu.VMEM_SHARED((n_rows, W), jnp.int32),
            pltpu.VMEM((n_chunks, C), jnp.int32),
            pltpu.VMEM((n_chunks, C), jnp.int32),
            pltpu.VMEM((C, W), jnp.int32),
            pltpu.VMEM((C, W), jnp.int32),
            pltpu.VMEM((C, W), jnp.int32),
            pltpu.VMEM((C, W), jnp.int32),
            pltpu.VMEM((per_w,), jnp.float32),
            pltpu.SemaphoreType.DMA,
            pltpu.SemaphoreType.DMA,
            pltpu.SemaphoreType.DMA,
            pltpu.SemaphoreType.DMA,
            pltpu.SemaphoreType.DMA,
        ],
    )
    def k(h_hbm, src_hbm, dst_hbm, out_hbm,
          h_sh, sslab, dslab, srows0, drows0, srows1, drows1, obuf,
          sem_i, sem_s0, sem_d0, sem_s1, sem_d1):
        cid = lax.axis_index("c")
        sid = lax.axis_index("s")
        wid = cid * NS + sid
        base = wid * per_w

        ci1 = pltpu.async_copy(src_hbm.at[wid], sslab, sem_i)
        ci2 = pltpu.async_copy(dst_hbm.at[wid], dslab, sem_i)

        @pl.when(sid < NS - 1)
        def _():
            r0 = sid * rows_per_tile
            pltpu.sync_copy(h_hbm.at[pl.ds(r0, rows_per_tile)],
                            h_sh.at[pl.ds(r0, rows_per_tile)])

        @pl.when(sid == NS - 1)
        def _():
            r0 = (NS - 1) * rows_per_tile
            pltpu.sync_copy(h_hbm.at[pl.ds(r0, tail_rows)],
                            h_sh.at[pl.ds(r0, tail_rows)])

        ci1.wait()
        ci2.wait()
        plsc.subcore_barrier()

        bufs = ((srows0, drows0, sem_s0, sem_d0),
                (srows1, drows1, sem_s1, sem_d1))

        def issue(t, b):
            sr, dr, ss, sd = bufs[b]
            pltpu.async_copy(h_sh.at[sslab.at[t]], sr, ss)
            pltpu.async_copy(h_sh.at[dslab.at[t]], dr, sd)

        def drain(t, b):
            sr, dr, ss, sd = bufs[b]
            pltpu.make_async_copy(h_sh.at[sslab.at[t]], sr, ss).wait()
            pltpu.make_async_copy(h_sh.at[dslab.at[t]], dr, sd).wait()

        def compute(t, b):
            sr, dr = bufs[b][0], bufs[b][1]
            lane = lax.iota(jnp.int32, L)
            last = lane == (L - 1)

            @pl.loop(0, C // L)
            def _grp(g):
                o0 = t * C + g * L
                o0v = jnp.full((L,), 0, jnp.int32) + o0
                for j in range(L):
                    e = g * L + j
                    p = None
                    for kk in range(W // L):
                        sv = plsc.bitcast(sr[e, pl.ds(kk * L, L)],
                                          jnp.bfloat16)
                        dv = plsc.bitcast(dr[e, pl.ds(kk * L, L)],
                                          jnp.bfloat16)
                        sa, sb = plsc.unpack(
                            sv, format=plsc.PackFormat.INTERLEAVED)
                        da, db = plsc.unpack(
                            dv, format=plsc.PackFormat.INTERLEAVED)
                        q = sa * da + sb * db
                        p = q if p is None else p + q
                    ps = lax.cumsum(p, axis=0)
                    plsc.store_scatter(obuf, [o0v + j], ps, mask=last)

        issue(0, 0)

        @pl.loop(0, n_chunks, step=2)
        def _chunk(t):
            issue(t + 1, 1)
            drain(t, 0)
            compute(t, 0)

            @pl.when(t + 2 < n_chunks)
            def _():
                issue(t + 2, 0)

            drain(t + 1, 1)
            compute(t + 1, 1)

        pltpu.sync_copy(obuf, out_hbm.at[pl.ds(base, per_w)])

    return k


def kernel(h, edge_index):
    E = edge_index.shape[1]
    src = edge_index[0].astype(jnp.int32)
    dst = edge_index[1].astype(jnp.int32)

    step = NW * C * 2
    E_pad = ((E + step - 1) // step) * step
    if E_pad != E:
        pad = E_pad - E
        zeros = jnp.zeros((pad,), jnp.int32)
        src = jnp.concatenate([src, zeros])
        dst = jnp.concatenate([dst, zeros])

    per_w = E_pad // NW
    src = src.reshape(NW, per_w // C, C)
    dst = dst.reshape(NW, per_w // C, C)

    h32 = jax.lax.bitcast_convert_type(
        h.astype(jnp.bfloat16).reshape(h.shape[0], W, 2), jnp.int32)
    out = _dot_kernel(E_pad, h.shape[0])(h32, src, dst)
    return out[:E].reshape(E, 1)


# speedup vs baseline: 2.7392x; 1.0171x over previous
import dataclasses
import functools

import jax
import jax.numpy as jnp
from jax import lax
from jax.experimental import pallas as pl
from jax.experimental.pallas import tpu as pltpu
from jax.experimental.pallas import tpu_sc as plsc

D = 128
W = D // 2
L = 16
NC, NS = 2, 16
NW = NC * NS
C = 128


@functools.cache
def _dot_kernel(E_pad, n_rows):
    per_w = E_pad // NW
    n_chunks = per_w // C
    assert n_chunks % 2 == 0 and n_chunks * C * NW == E_pad

    rows_per_tile = ((n_rows // NS) // 8) * 8
    tail_rows = n_rows - (NS - 1) * rows_per_tile

    mesh = plsc.VectorSubcoreMesh(core_axis_name="c", subcore_axis_name="s")

    cp = pltpu.CompilerParams(use_tc_tiling_on_sc=False)
    if "needs_layout_passes" in pltpu.CompilerParams.__dataclass_fields__:
        cp = dataclasses.replace(cp, needs_layout_passes=False)

    @functools.partial(
        pl.kernel,
        mesh=mesh,
        compiler_params=cp,
        out_type=jax.ShapeDtypeStruct((E_pad,), jnp.float32),
        scratch_types=[
            pltpu.VMEM_SHARED((n_rows, W), jnp.int32),
            pltpu.VMEM((n_chunks, C), jnp.int32),
            pltpu.VMEM((n_chunks, C), jnp.int32),
            pltpu.VMEM((C, W), jnp.int32),
            pltpu.VMEM((C, W), jnp.int32),
            pltpu.VMEM((C, W), jnp.int32),
            pltpu.VMEM((C, W), jnp.int32),
            pltpu.VMEM((per_w,), jnp.float32),
            pltpu.SemaphoreType.DMA,
            pltpu.SemaphoreType.DMA,
            pltpu.SemaphoreType.DMA,
            pltpu.SemaphoreType.DMA,
            pltpu.SemaphoreType.DMA,
        ],
    )
    def k(h_hbm, src_hbm, dst_hbm, out_hbm,
          h_sh, sslab, dslab, srows0, drows0, srows1, drows1, obuf,
          sem_i, sem_s0, sem_d0, sem_s1, sem_d1):
        cid = lax.axis_index("c")
        sid = lax.axis_index("s")
        wid = cid * NS + sid
        base = wid * per_w

        ci1 = pltpu.async_copy(src_hbm.at[wid], sslab, sem_i)
        ci2 = pltpu.async_copy(dst_hbm.at[wid], dslab, sem_i)

        @pl.when(sid < NS - 1)
        def _():
            r0 = sid * rows_per_tile
            pltpu.sync_copy(h_hbm.at[pl.ds(r0, rows_per_tile)],
                            h_sh.at[pl.ds(r0, rows_per_tile)])

        @pl.when(sid == NS - 1)
        def _():
            r0 = (NS - 1) * rows_per_tile
            pltpu.sync_copy(h_hbm.at[pl.ds(r0, tail_rows)],
                            h_sh.at[pl.ds(r0, tail_rows)])

        ci1.wait()
        ci2.wait()
        plsc.subcore_barrier()

        bufs = ((srows0, drows0, sem_s0, sem_d0),
                (srows1, drows1, sem_s1, sem_d1))

        def issue(t, b):
            sr, dr, ss, sd = bufs[b]
            pltpu.async_copy(h_sh.at[sslab.at[t]], sr, ss)
            pltpu.async_copy(h_sh.at[dslab.at[t]], dr, sd)

        def drain(t, b):
            sr, dr, ss, sd = bufs[b]
            pltpu.make_async_copy(h_sh.at[sslab.at[t]], sr, ss).wait()
            pltpu.make_async_copy(h_sh.at[dslab.at[t]], dr, sd).wait()

        def compute(t, b):
            sr, dr = bufs[b][0], bufs[b][1]
            lane = lax.iota(jnp.int32, L)
            last = lane == (L - 1)

            @plsc.parallel_loop(0, C // L, unroll=2)
            def _grp(g):
                o0 = t * C + g * L
                o0v = jnp.full((L,), 0, jnp.int32) + o0
                for j in range(L):
                    e = g * L + j
                    p = None
                    for kk in range(W // L):
                        sv = plsc.bitcast(sr[e, pl.ds(kk * L, L)],
                                          jnp.bfloat16)
                        dv = plsc.bitcast(dr[e, pl.ds(kk * L, L)],
                                          jnp.bfloat16)
                        sa, sb = plsc.unpack(
                            sv, format=plsc.PackFormat.INTERLEAVED)
                        da, db = plsc.unpack(
                            dv, format=plsc.PackFormat.INTERLEAVED)
                        q = sa * da + sb * db
                        p = q if p is None else p + q
                    ps = lax.cumsum(p, axis=0)
                    plsc.store_scatter(obuf, [o0v + j], ps, mask=last)

        issue(0, 0)

        @pl.loop(0, n_chunks, step=2)
        def _chunk(t):
            issue(t + 1, 1)
            drain(t, 0)
            compute(t, 0)

            @pl.when(t + 2 < n_chunks)
            def _():
                issue(t + 2, 0)

            drain(t + 1, 1)
            compute(t + 1, 1)

        pltpu.sync_copy(obuf, out_hbm.at[pl.ds(base, per_w)])

    return k


def kernel(h, edge_index):
    E = edge_index.shape[1]
    src = edge_index[0].astype(jnp.int32)
    dst = edge_index[1].astype(jnp.int32)

    step = NW * C * 2
    E_pad = ((E + step - 1) // step) * step
    if E_pad != E:
        pad = E_pad - E
        zeros = jnp.zeros((pad,), jnp.int32)
        src = jnp.concatenate([src, zeros])
        dst = jnp.concatenate([dst, zeros])

    per_w = E_pad // NW
    src = src.reshape(NW, per_w // C, C)
    dst = dst.reshape(NW, per_w // C, C)

    h32 = jax.lax.bitcast_convert_type(
        h.astype(jnp.bfloat16).reshape(h.shape[0], W, 2), jnp.int32)
    out = _dot_kernel(E_pad, h.shape[0])(h32, src, dst)
    return out[:E].reshape(E, 1)
